# baseline (speedup 1.0000x reference)
import functools

import jax
import jax.numpy as jnp
from jax import lax
from jax.experimental import pallas as pl
from jax.experimental.pallas import tpu as pltpu
from jax.experimental.pallas import tpu_sc as plsc

N = 10000
NPAD = 10240
E = 320000
DIM = 128
HEADS = 8
HD = 16
SCALE = HD ** -0.5

NC = 2
NS = 16
NW = NC * NS
CB = 128
NCH = E // CB
CH_PER_W = -(-NCH // NW)
CB3 = 40
EPW = E // NW
CH3 = EPW // CB3

_f32 = jnp.float32
_mesh = plsc.VectorSubcoreMesh(core_axis_name="c", subcore_axis_name="s")


def _matmul_t(a, w):
    return lax.dot_general(a, w, (((1,), (1,)), ((), ())),
                           preferred_element_type=_f32)


def _node_proj_body(x_ref, wq_ref, wk_ref, wv_ref, q_ref, k_ref, v_ref):
    xb = x_ref[...]
    q_ref[...] = _matmul_t(xb, wq_ref[...])
    k_ref[...] = _matmul_t(xb, wk_ref[...])
    v_ref[...] = _matmul_t(xb, wv_ref[...])


def _tc_node_proj(x, wq_s, wk, wv):
    bn = 2000
    grid = (N // bn,)
    bspec = pl.BlockSpec((bn, DIM), lambda i: (i, 0))
    wspec = pl.BlockSpec((DIM, DIM), lambda i: (0, 0))
    out = jax.ShapeDtypeStruct((N, DIM), _f32)
    return pl.pallas_call(
        _node_proj_body,
        grid=grid,
        in_specs=[bspec, wspec, wspec, wspec],
        out_specs=[bspec, bspec, bspec],
        out_shape=[out, out, out],
    )(x, wq_s, wk, wv)


def _edge_proj_body(e_ref, w_ref, o_ref):
    o_ref[...] = _matmul_t(e_ref[...], w_ref[...])


def _tc_edge_proj1(edges, w):
    be = 10000
    grid = (E // be,)
    bspec = pl.BlockSpec((be, DIM), lambda i: (i, 0))
    wspec = pl.BlockSpec((DIM, DIM), lambda i: (0, 0))
    return pl.pallas_call(
        _edge_proj_body,
        grid=grid,
        in_specs=[bspec, wspec],
        out_specs=bspec,
        out_shape=jax.ShapeDtypeStruct((E, DIM), _f32),
    )(edges, w)


def _head_mlp_body(g_ref, bexp_ref, bsq_ref, ex_ref):
    g = g_ref[...]
    g = jnp.dot(g, bexp_ref[...], preferred_element_type=_f32)
    g = 0.5 * g * (1.0 + lax.erf(g * (2.0 ** -0.5)))
    g = jnp.dot(g, bsq_ref[...], preferred_element_type=_f32)
    ex_ref[...] = jnp.exp(g)


def _tc_head_mlp(g_packed, bexp, bsq):
    rows = E // 8
    br = 2000
    grid = (rows // br,)
    bspec = pl.BlockSpec((br, DIM), lambda i: (i, 0))
    wspec = pl.BlockSpec((DIM, DIM), lambda i: (0, 0))
    return pl.pallas_call(
        _head_mlp_body,
        grid=grid,
        in_specs=[bspec, wspec, wspec],
        out_specs=bspec,
        out_shape=jax.ShapeDtypeStruct((rows, DIM), _f32),
    )(g_packed, bexp, bsq)


def _out_proj_body(p_ref, w_ref, b_ref, o_ref):
    acc = p_ref[0] + p_ref[1]
    o_ref[...] = _matmul_t(acc, w_ref[...]) + b_ref[...]


def _tc_out_proj(parts, wout, bout2d):
    bn = 2000
    grid = (N // bn,)
    return pl.pallas_call(
        _out_proj_body,
        grid=grid,
        in_specs=[
            pl.BlockSpec((2, bn, DIM), lambda i: (0, i, 0)),
            pl.BlockSpec((DIM, DIM), lambda i: (0, 0)),
            pl.BlockSpec((1, DIM), lambda i: (0, 0)),
        ],
        out_specs=pl.BlockSpec((bn, DIM), lambda i: (i, 0)),
        out_shape=jax.ShapeDtypeStruct((N, DIM), _f32),
    )(parts, wout, bout2d)


CB1 = 64
NCH1 = E // CB1
CH1 = 158


def _sc_logits_kernel(qn_hbm, kn_hbm, ek_hbm, src_hbm, dst_hbm, out_hbm,
                      srcv, dstv, qrows, krows, ekrows, lg, fbuf,
                      sq0, sq1, sk0, sk1, se0, se1,
                      ss0, ss1, sd0, sd1, so0, so1):
    w = lax.axis_index("s") * NC + lax.axis_index("c")
    l16 = lax.iota(jnp.int32, HD)
    i0 = ((l16 >> 3) * 16) + (l16 & 7)
    i1 = 128 + ((l16 >> 2) * 8) + (l16 & 3)
    i2 = 192 + ((l16 >> 1) * 4) + (l16 & 1)
    i3 = 224 + ((l16 & 7) * 2)
    sq = (sq0, sq1)
    sk = (sk0, sk1)
    se = (se0, se1)
    ss = (ss0, ss1)
    sd = (sd0, sd1)
    so = (so0, so1)

    def cbase(i):
        return lax.rem(w + i * NW, NCH1) * CB1

    def issue_idx(i, b):
        base = cbase(i)
        pltpu.async_copy(src_hbm.at[pl.ds(base, CB1)], srcv.at[b], ss[b])
        pltpu.async_copy(dst_hbm.at[pl.ds(base, CB1)], dstv.at[b], sd[b])

    def wait_idx(b):
        pltpu.make_async_copy(src_hbm.at[pl.ds(0, CB1)], srcv.at[b], ss[b]).wait()
        pltpu.make_async_copy(dst_hbm.at[pl.ds(0, CB1)], dstv.at[b], sd[b]).wait()

    def issue_gathers(i, b):
        base = cbase(i)
        pltpu.async_copy(qn_hbm.at[srcv.at[b]], qrows.at[b], sq[b])
        pltpu.async_copy(kn_hbm.at[dstv.at[b]], krows.at[b], sk[b])
        pltpu.async_copy(ek_hbm.at[pl.ds(base, CB1)], ekrows.at[b], se[b])

    def wait_gathers(b):
        pltpu.make_async_copy(qn_hbm.at[pl.ds(0, CB1)], qrows.at[b], sq[b]).wait()
        pltpu.make_async_copy(kn_hbm.at[pl.ds(0, CB1)], krows.at[b], sk[b]).wait()
        pltpu.make_async_copy(ek_hbm.at[pl.ds(0, CB1)], ekrows.at[b], se[b]).wait()

    def wait_out(b):
        pltpu.make_async_copy(lg.at[b], out_hbm.at[pl.ds(0, CB1)], so[b]).wait()

    def compute(i, b):
        qr = qrows.at[b]
        kr = krows.at[b]
        er = ekrows.at[b]
        lgb = lg.at[b]

        def edge_body(t, carry2):
            for u in range(4):
                e = t * 4 + u
                fb = u * 256
                for h in range(HEADS):
                    sl = pl.ds(h * HD, HD)
                    fbuf[pl.ds(fb + h * HD, HD)] = (
                        qr[e, sl] * (kr[e, sl] + er[e, sl]))
            for u in range(4):
                fb = u * 256
                for g in range(4):
                    v = (plsc.load_gather(fbuf, [fb + 32 * g + i0])
                         + plsc.load_gather(fbuf, [fb + 32 * g + i0 + 8]))
                    fbuf[pl.ds(fb + 128 + 16 * g, HD)] = v
                for m in range(2):
                    v = (plsc.load_gather(fbuf, [fb + 32 * m + i1])
                         + plsc.load_gather(fbuf, [fb + 32 * m + i1 + 4]))
                    fbuf[pl.ds(fb + 192 + 16 * m, HD)] = v
                v = (plsc.load_gather(fbuf, [fb + i2])
                     + plsc.load_gather(fbuf, [fb + i2 + 2]))
                fbuf[pl.ds(fb + 224, HD)] = v
                lgb[t * 4 + u] = (plsc.load_gather(fbuf, [fb + i3])
                                  + plsc.load_gather(fbuf, [fb + i3 + 1]))
            return carry2

        lax.fori_loop(0, CB1 // 4, edge_body, 0)
        pltpu.async_copy(lgb, out_hbm.at[pl.ds(cbase(i), CB1)], so[b])

    issue_idx(0, 0)
    issue_idx(1, 1)
    wait_idx(0)
    issue_gathers(0, 0)

    def body2(i2, carry):
        for b in (0, 1):
            i = i2 * 2 + b
            wait_gathers(b)
            issue_idx(i + 2, b)
            wait_idx(1 - b)
            issue_gathers(i + 1, 1 - b)

            @pl.when(i >= 2)
            def _():
                wait_out(b)

            compute(i, b)
        return carry

    lax.fori_loop(0, CH1 // 2, body2, 0)
    wait_gathers(0)
    wait_idx(1)
    wait_out(0)
    wait_out(1)


def _sc_logits(qn, kn, ek, src, dst):
    kern = pl.kernel(
        _sc_logits_kernel,
        out_type=jax.ShapeDtypeStruct((E, HD), _f32),
        mesh=_mesh,
        compiler_params=pltpu.CompilerParams(needs_layout_passes=False),
        scratch_types=[
            pltpu.VMEM((2, CB1), jnp.int32),
            pltpu.VMEM((2, CB1), jnp.int32),
            pltpu.VMEM((2, CB1, DIM), _f32),
            pltpu.VMEM((2, CB1, DIM), _f32),
            pltpu.VMEM((2, CB1, DIM), _f32),
            pltpu.VMEM((2, CB1, HD), _f32),
            pltpu.VMEM((1024,), _f32),
        ] + [pltpu.SemaphoreType.DMA] * 12,
    )
    return kern(qn, kn, ek, src, dst)


def _sc_segsum_kernel(ex_hbm, src_hbm, out_hbm, srcv, exrows, padbuf, shared, sem):
    cid = lax.axis_index("c")
    sid = lax.axis_index("s")
    w = sid * NC + cid
    rows_per_sub = NPAD // NS

    def zrow(r, carry):
        for j in range(DIM // HD):
            padbuf[r, pl.ds(j * HD, HD)] = jnp.zeros((HD,), _f32)
        return carry

    lax.fori_loop(0, CB, zrow, 0)
    for t in range(rows_per_sub // CB):
        pltpu.sync_copy(padbuf, shared.at[pl.ds(sid * rows_per_sub + t * CB, CB)])
    plsc.subcore_barrier()

    def chunk_body(i, carry):
        c = w + i * NW

        @pl.when(c < NCH)
        def _():
            base = c * CB
            pltpu.sync_copy(src_hbm.at[pl.ds(base, CB)], srcv)
            pltpu.sync_copy(ex_hbm.at[pl.ds(base, CB)], exrows)

            def fill(e, carry2):
                padbuf[e, pl.ds(0, HD)] = exrows[e]
                return carry2

            lax.fori_loop(0, CB, fill, 0)
            pltpu.sync_copy(padbuf, shared.at[srcv], add=True)

        return carry

    lax.fori_loop(0, CH_PER_W, chunk_body, 0)
    plsc.subcore_barrier()
    for t in range(rows_per_sub // CB):
        off = sid * rows_per_sub + t * CB
        pltpu.sync_copy(shared.at[pl.ds(off, CB)], out_hbm.at[cid].at[pl.ds(off, CB)])


def _sc_segsum(ex16, src):
    kern = pl.kernel(
        _sc_segsum_kernel,
        out_type=jax.ShapeDtypeStruct((NC, NPAD, DIM), _f32),
        mesh=_mesh,
        compiler_params=pltpu.CompilerParams(needs_layout_passes=False),
        scratch_types=[
            pltpu.VMEM((CB,), jnp.int32),
            pltpu.VMEM((CB, HD), _f32),
            pltpu.VMEM((CB, DIM), _f32),
            pltpu.VMEM_SHARED((NPAD, DIM), _f32),
            pltpu.SemaphoreType.DMA,
        ],
    )
    return kern(ex16, src)


def _combine_body(p_ref, o_ref):
    o_ref[...] = p_ref[0] + p_ref[1]


def _tc_combine(parts):
    bn = 2048
    grid = (NPAD // bn,)
    return pl.pallas_call(
        _combine_body,
        grid=grid,
        in_specs=[pl.BlockSpec((2, bn, DIM), lambda i: (0, i, 0))],
        out_specs=pl.BlockSpec((bn, DIM), lambda i: (i, 0)),
        out_shape=jax.ShapeDtypeStruct((NPAD, DIM), _f32),
    )(parts)


def _sc_aggregate_kernel(vn_hbm, ev_hbm, ex_hbm, ssum_hbm, src_hbm, dst_hbm,
                         outp_hbm,
                         srcv, dstv, vrows, evrows, exrows, shared,
                         sv0, sv1, se0, se1, sx0, sx1, ss0, ss1, sd0, sd1):
    cid = lax.axis_index("c")
    sid = lax.axis_index("s")
    w = sid * NC + cid
    rows_per_sub = NPAD // NS
    sv = (sv0, sv1)
    se = (se0, se1)
    sx = (sx0, sx1)
    ss = (ss0, ss1)
    sd = (sd0, sd1)

    def zrow(r, carry):
        for j in range(DIM // HD):
            vrows[0, r, pl.ds(j * HD, HD)] = jnp.zeros((HD,), _f32)
        return carry

    lax.fori_loop(0, CB3, zrow, 0)
    for t in range(rows_per_sub // CB3):
        pltpu.sync_copy(vrows.at[0], shared.at[pl.ds(sid * rows_per_sub + t * CB3, CB3)])
    plsc.subcore_barrier()

    def cbase(i):
        return w * EPW + jnp.minimum(i, CH3 - 1) * CB3

    def issue_idx(i, b):
        base = cbase(i)
        pltpu.async_copy(src_hbm.at[pl.ds(base, CB3)], srcv.at[b], ss[b])
        pltpu.async_copy(dst_hbm.at[pl.ds(base, CB3)], dstv.at[b], sd[b])

    def wait_idx(b):
        pltpu.make_async_copy(src_hbm.at[pl.ds(0, CB3)], srcv.at[b], ss[b]).wait()
        pltpu.make_async_copy(dst_hbm.at[pl.ds(0, CB3)], dstv.at[b], sd[b]).wait()

    def issue_gathers(i, b):
        base = cbase(i)
        pltpu.async_copy(vn_hbm.at[dstv.at[b]], vrows.at[b], sv[b])
        pltpu.async_copy(ev_hbm.at[pl.ds(base, CB3)], evrows.at[b], se[b])
        pltpu.async_copy(ex_hbm.at[pl.ds(base, CB3)], exrows.at[b], sx[b])

    def wait_gathers(b):
        pltpu.make_async_copy(vn_hbm.at[pl.ds(0, CB3)], vrows.at[b], sv[b]).wait()
        pltpu.make_async_copy(ev_hbm.at[pl.ds(0, CB3)], evrows.at[b], se[b]).wait()
        pltpu.make_async_copy(ex_hbm.at[pl.ds(0, CB3)], exrows.at[b], sx[b]).wait()

    def compute(i, b):
        vr = vrows.at[b]
        er = evrows.at[b]
        xr = exrows.at[b]

        def edge_body(t, carry2):
            for u in range(2):
                e = t * 2 + u
                exv = xr[e]
                for h in range(HEADS):
                    a = exv[h]
                    sl = pl.ds(h * HD, HD)
                    vr[e, sl] = (vr[e, sl] + er[e, sl]) * a
            return carry2

        lax.fori_loop(0, CB3 // 2, edge_body, 0)
        pltpu.sync_copy(vr, shared.at[srcv.at[b]], add=True)

    issue_idx(0, 0)
    issue_idx(1, 1)
    wait_idx(0)
    issue_gathers(0, 0)

    def body2(i2, carry):
        for b in (0, 1):
            i = i2 * 2 + b
            wait_idx(1 - b)
            issue_gathers(i + 1, 1 - b)
            wait_gathers(b)
            compute(i, b)
            issue_idx(i + 2, b)
        return carry

    lax.fori_loop(0, CH3 // 2, body2, 0)
    wait_gathers(0)
    wait_idx(1)
    plsc.subcore_barrier()
    for t in range(rows_per_sub // CB3):
        off = sid * rows_per_sub + t * CB3
        pltpu.sync_copy(shared.at[pl.ds(off, CB3)], vrows.at[0])
        pltpu.sync_copy(ssum_hbm.at[pl.ds(off, CB3)], evrows.at[0])

        def nrow(r, carry):
            rec = 1.0 / (evrows[0, r, pl.ds(0, HD)] + 1e-16)
            for h in range(HEADS):
                sl = pl.ds(h * HD, HD)
                vrows[0, r, sl] = vrows[0, r, sl] * rec[h]
            return carry

        lax.fori_loop(0, CB3, nrow, 0)
        pltpu.sync_copy(vrows.at[0], outp_hbm.at[cid].at[pl.ds(off, CB3)])


def _sc_aggregate(vn, ev, ex16, ssum, src, dst):
    kern = pl.kernel(
        _sc_aggregate_kernel,
        out_type=jax.ShapeDtypeStruct((NC, NPAD, DIM), _f32),
        mesh=_mesh,
        compiler_params=pltpu.CompilerParams(needs_layout_passes=False),
        scratch_types=[
            pltpu.VMEM((2, CB3), jnp.int32),
            pltpu.VMEM((2, CB3), jnp.int32),
            pltpu.VMEM((2, CB3, DIM), _f32),
            pltpu.VMEM((2, CB3, DIM), _f32),
            pltpu.VMEM((2, CB3, HD), _f32),
            pltpu.VMEM_SHARED((NPAD, DIM), _f32),
        ] + [pltpu.SemaphoreType.DMA] * 10,
    )
    return kern(vn, ev, ex16, ssum, src, dst)


CBA = 128
NCHA = E // CBA
CHA = 80


def _sc_attn_kernel(ex_hbm, ssum_hbm, src_hbm, attn_hbm,
                    srcv, srows, exrows, attn_st,
                    ss0, ss1, sr0, sr1, sx0, sx1, so0, so1):
    w = lax.axis_index("s") * NC + lax.axis_index("c")
    ss = (ss0, ss1)
    sr = (sr0, sr1)
    sx = (sx0, sx1)
    so = (so0, so1)

    def cbase(i):
        return lax.rem(w + i * NW, NCHA) * CBA

    def issue_idx(i, b):
        pltpu.async_copy(src_hbm.at[pl.ds(cbase(i), CBA)], srcv.at[b], ss[b])

    def wait_idx(b):
        pltpu.make_async_copy(src_hbm.at[pl.ds(0, CBA)], srcv.at[b], ss[b]).wait()

    def issue_gathers(i, b):
        pltpu.async_copy(ssum_hbm.at[srcv.at[b]], srows.at[b], sr[b])
        pltpu.async_copy(ex_hbm.at[pl.ds(cbase(i), CBA)], exrows.at[b], sx[b])

    def wait_gathers(b):
        pltpu.make_async_copy(ssum_hbm.at[pl.ds(0, CBA)], srows.at[b], sr[b]).wait()
        pltpu.make_async_copy(ex_hbm.at[pl.ds(0, CBA)], exrows.at[b], sx[b]).wait()

    def wait_out(b):
        pltpu.make_async_copy(attn_st.at[b], attn_hbm.at[pl.ds(0, CBA)], so[b]).wait()

    def compute(i, b):
        sb = srows.at[b]
        xb = exrows.at[b]
        ab = attn_st.at[b]

        def edge_body(e, carry2):
            ab[e] = xb[e] / (sb[e, pl.ds(0, HD)] + 1e-16)
            return carry2

        lax.fori_loop(0, CBA, edge_body, 0)
        pltpu.async_copy(ab, attn_hbm.at[pl.ds(cbase(i), CBA)], so[b])

    issue_idx(0, 0)
    issue_idx(1, 1)
    wait_idx(0)
    issue_gathers(0, 0)

    def body2(i2, carry):
        for b in (0, 1):
            i = i2 * 2 + b
            wait_gathers(b)
            issue_idx(i + 2, b)
            wait_idx(1 - b)
            issue_gathers(i + 1, 1 - b)

            @pl.when(i >= 2)
            def _():
                wait_out(b)

            compute(i, b)
        return carry

    lax.fori_loop(0, CHA // 2, body2, 0)
    wait_gathers(0)
    wait_idx(1)
    wait_out(0)
    wait_out(1)


def _sc_attn(ex16, ssum, src):
    kern = pl.kernel(
        _sc_attn_kernel,
        out_type=jax.ShapeDtypeStruct((E, HD), _f32),
        mesh=_mesh,
        compiler_params=pltpu.CompilerParams(needs_layout_passes=False),
        scratch_types=[
            pltpu.VMEM((2, CBA), jnp.int32),
            pltpu.VMEM((2, CBA, DIM), _f32),
            pltpu.VMEM((2, CBA, HD), _f32),
            pltpu.VMEM((2, CBA, HD), _f32),
        ] + [pltpu.SemaphoreType.DMA] * 8,
    )
    return kern(ex16, ssum, src)


def kernel(x, edges, edge_index, Wq, Wk, Wv, Wek, Wev, Wexp, Wsq, Wout, bout):
    src = edge_index[0]
    dst = edge_index[1]

    wq_s = Wq * SCALE
    pexp = jnp.zeros((HD, HD), _f32).at[:HEADS, :HEADS].set(Wexp.T.astype(_f32))
    psq = jnp.zeros((HD, HD), _f32).at[:HEADS, :HEADS].set(Wsq.T.astype(_f32))
    eye8 = jnp.eye(HEADS, dtype=_f32)
    bexp = jnp.kron(eye8, pexp)
    bsq = jnp.kron(eye8, psq)

    qn, kn, vn = _tc_node_proj(x, wq_s, Wk, Wv)
    ek = _tc_edge_proj1(edges, Wek)
    ev = _tc_edge_proj1(edges, Wev)

    logits = _sc_logits(qn, kn, ek, src, dst)
    g_packed = logits.reshape(E // 8, DIM)
    ex_packed = _tc_head_mlp(g_packed, bexp, bsq)
    ex16 = ex_packed.reshape(E, HD)

    ssum_p = _sc_segsum(ex16, src)
    ssum = _tc_combine(ssum_p)
    attn16 = _sc_attn(ex16, ssum, src)
    out_p = _sc_aggregate(vn, ev, ex16, ssum, src, dst)

    out = _tc_out_proj(out_p, Wout, bout.reshape(1, DIM))
    attn_he = attn16[:, :HEADS].T
    return out, attn_he


# speedup vs baseline: 1.0506x; 1.0506x over previous
import functools

import jax
import jax.numpy as jnp
from jax import lax
from jax.experimental import pallas as pl
from jax.experimental.pallas import tpu as pltpu
from jax.experimental.pallas import tpu_sc as plsc

N = 10000
NPAD = 10240
E = 320000
DIM = 128
HEADS = 8
HD = 16
SCALE = HD ** -0.5

NC = 2
NS = 16
NW = NC * NS
CB = 128
NCH = E // CB
CH_PER_W = -(-NCH // NW)
CB3 = 40
EPW = E // NW
CH3 = EPW // CB3

_f32 = jnp.float32
_mesh = plsc.VectorSubcoreMesh(core_axis_name="c", subcore_axis_name="s")


def _matmul_t(a, w):
    return lax.dot_general(a, w, (((1,), (1,)), ((), ())),
                           preferred_element_type=_f32)


def _node_proj_body(x_ref, wq_ref, wk_ref, wv_ref, q_ref, k_ref, v_ref):
    xb = x_ref[...]
    q_ref[...] = _matmul_t(xb, wq_ref[...])
    k_ref[...] = _matmul_t(xb, wk_ref[...])
    v_ref[...] = _matmul_t(xb, wv_ref[...])


def _tc_node_proj(x, wq_s, wk, wv):
    bn = 2000
    grid = (N // bn,)
    bspec = pl.BlockSpec((bn, DIM), lambda i: (i, 0))
    wspec = pl.BlockSpec((DIM, DIM), lambda i: (0, 0))
    out = jax.ShapeDtypeStruct((N, DIM), _f32)
    return pl.pallas_call(
        _node_proj_body,
        grid=grid,
        in_specs=[bspec, wspec, wspec, wspec],
        out_specs=[bspec, bspec, bspec],
        out_shape=[out, out, out],
    )(x, wq_s, wk, wv)


def _edge_proj_body(e_ref, wek_ref, wev_ref, ek_ref, ev_ref):
    eb = e_ref[...]
    ek_ref[...] = _matmul_t(eb, wek_ref[...])
    ev_ref[...] = _matmul_t(eb, wev_ref[...])


def _tc_edge_proj(edges, wek, wev):
    be = 10000
    grid = (E // be,)
    bspec = pl.BlockSpec((be, DIM), lambda i: (i, 0))
    wspec = pl.BlockSpec((DIM, DIM), lambda i: (0, 0))
    out = jax.ShapeDtypeStruct((E, DIM), _f32)
    return pl.pallas_call(
        _edge_proj_body,
        grid=grid,
        in_specs=[bspec, wspec, wspec],
        out_specs=[bspec, bspec],
        out_shape=[out, out],
    )(edges, wek, wev)


def _head_mlp_body(g_ref, bexp_ref, bsq_ref, ex_ref):
    g = g_ref[...]
    g = jnp.dot(g, bexp_ref[...], preferred_element_type=_f32)
    g = 0.5 * g * (1.0 + lax.erf(g * (2.0 ** -0.5)))
    g = jnp.dot(g, bsq_ref[...], preferred_element_type=_f32)
    ex_ref[...] = jnp.exp(g)


def _tc_head_mlp(g_packed, bexp, bsq):
    rows = E // 8
    br = 2000
    grid = (rows // br,)
    bspec = pl.BlockSpec((br, DIM), lambda i: (i, 0))
    wspec = pl.BlockSpec((DIM, DIM), lambda i: (0, 0))
    return pl.pallas_call(
        _head_mlp_body,
        grid=grid,
        in_specs=[bspec, wspec, wspec],
        out_specs=bspec,
        out_shape=jax.ShapeDtypeStruct((rows, DIM), _f32),
    )(g_packed, bexp, bsq)


def _out_proj_body(p_ref, w_ref, b_ref, o_ref):
    acc = p_ref[0] + p_ref[1]
    o_ref[...] = _matmul_t(acc, w_ref[...]) + b_ref[...]


def _tc_out_proj(parts, wout, bout2d):
    bn = 2000
    grid = (N // bn,)
    return pl.pallas_call(
        _out_proj_body,
        grid=grid,
        in_specs=[
            pl.BlockSpec((2, bn, DIM), lambda i: (0, i, 0)),
            pl.BlockSpec((DIM, DIM), lambda i: (0, 0)),
            pl.BlockSpec((1, DIM), lambda i: (0, 0)),
        ],
        out_specs=pl.BlockSpec((bn, DIM), lambda i: (i, 0)),
        out_shape=jax.ShapeDtypeStruct((N, DIM), _f32),
    )(parts, wout, bout2d)


CB1 = 64
NCH1 = E // CB1
CH1 = 158


def _sc_logits_kernel(qn_hbm, kn_hbm, ek_hbm, src_hbm, dst_hbm, out_hbm,
                      srcv, dstv, qrows, krows, ekrows, lg, fbuf,
                      sq0, sq1, sk0, sk1, se0, se1,
                      ss0, ss1, sd0, sd1, so0, so1):
    w = lax.axis_index("s") * NC + lax.axis_index("c")
    l16 = lax.iota(jnp.int32, HD)
    i0 = ((l16 >> 3) * 16) + (l16 & 7)
    i1 = 128 + ((l16 >> 2) * 8) + (l16 & 3)
    i2 = 192 + ((l16 >> 1) * 4) + (l16 & 1)
    i3 = 224 + ((l16 & 7) * 2)
    sq = (sq0, sq1)
    sk = (sk0, sk1)
    se = (se0, se1)
    ss = (ss0, ss1)
    sd = (sd0, sd1)
    so = (so0, so1)

    def cbase(i):
        return lax.rem(w + i * NW, NCH1) * CB1

    def issue_idx(i, b):
        base = cbase(i)
        pltpu.async_copy(src_hbm.at[pl.ds(base, CB1)], srcv.at[b], ss[b])
        pltpu.async_copy(dst_hbm.at[pl.ds(base, CB1)], dstv.at[b], sd[b])

    def wait_idx(b):
        pltpu.make_async_copy(src_hbm.at[pl.ds(0, CB1)], srcv.at[b], ss[b]).wait()
        pltpu.make_async_copy(dst_hbm.at[pl.ds(0, CB1)], dstv.at[b], sd[b]).wait()

    def issue_gathers(i, b):
        base = cbase(i)
        pltpu.async_copy(qn_hbm.at[srcv.at[b]], qrows.at[b], sq[b])
        pltpu.async_copy(kn_hbm.at[dstv.at[b]], krows.at[b], sk[b])
        pltpu.async_copy(ek_hbm.at[pl.ds(base, CB1)], ekrows.at[b], se[b])

    def wait_gathers(b):
        pltpu.make_async_copy(qn_hbm.at[pl.ds(0, CB1)], qrows.at[b], sq[b]).wait()
        pltpu.make_async_copy(kn_hbm.at[pl.ds(0, CB1)], krows.at[b], sk[b]).wait()
        pltpu.make_async_copy(ek_hbm.at[pl.ds(0, CB1)], ekrows.at[b], se[b]).wait()

    def wait_out(b):
        pltpu.make_async_copy(lg.at[b], out_hbm.at[pl.ds(0, CB1)], so[b]).wait()

    def compute(i, b):
        qr = qrows.at[b]
        kr = krows.at[b]
        er = ekrows.at[b]
        lgb = lg.at[b]

        def edge_body(t, carry2):
            for u in range(4):
                e = t * 4 + u
                fb = u * 256
                for h in range(HEADS):
                    sl = pl.ds(h * HD, HD)
                    fbuf[pl.ds(fb + h * HD, HD)] = (
                        qr[e, sl] * (kr[e, sl] + er[e, sl]))
            for u in range(4):
                fb = u * 256
                for g in range(4):
                    v = (plsc.load_gather(fbuf, [fb + 32 * g + i0])
                         + plsc.load_gather(fbuf, [fb + 32 * g + i0 + 8]))
                    fbuf[pl.ds(fb + 128 + 16 * g, HD)] = v
                for m in range(2):
                    v = (plsc.load_gather(fbuf, [fb + 32 * m + i1])
                         + plsc.load_gather(fbuf, [fb + 32 * m + i1 + 4]))
                    fbuf[pl.ds(fb + 192 + 16 * m, HD)] = v
                v = (plsc.load_gather(fbuf, [fb + i2])
                     + plsc.load_gather(fbuf, [fb + i2 + 2]))
                fbuf[pl.ds(fb + 224, HD)] = v
                lgb[t * 4 + u] = (plsc.load_gather(fbuf, [fb + i3])
                                  + plsc.load_gather(fbuf, [fb + i3 + 1]))
            return carry2

        lax.fori_loop(0, CB1 // 4, edge_body, 0)
        pltpu.async_copy(lgb, out_hbm.at[pl.ds(cbase(i), CB1)], so[b])

    issue_idx(0, 0)
    issue_idx(1, 1)
    wait_idx(0)
    issue_gathers(0, 0)

    def body2(i2, carry):
        for b in (0, 1):
            i = i2 * 2 + b
            wait_gathers(b)
            issue_idx(i + 2, b)
            wait_idx(1 - b)
            issue_gathers(i + 1, 1 - b)

            @pl.when(i >= 2)
            def _():
                wait_out(b)

            compute(i, b)
        return carry

    lax.fori_loop(0, CH1 // 2, body2, 0)
    wait_gathers(0)
    wait_idx(1)
    wait_out(0)
    wait_out(1)


def _sc_logits(qn, kn, ek, src, dst):
    kern = pl.kernel(
        _sc_logits_kernel,
        out_type=jax.ShapeDtypeStruct((E, HD), _f32),
        mesh=_mesh,
        compiler_params=pltpu.CompilerParams(needs_layout_passes=False),
        scratch_types=[
            pltpu.VMEM((2, CB1), jnp.int32),
            pltpu.VMEM((2, CB1), jnp.int32),
            pltpu.VMEM((2, CB1, DIM), _f32),
            pltpu.VMEM((2, CB1, DIM), _f32),
            pltpu.VMEM((2, CB1, DIM), _f32),
            pltpu.VMEM((2, CB1, HD), _f32),
            pltpu.VMEM((1024,), _f32),
        ] + [pltpu.SemaphoreType.DMA] * 12,
    )
    return kern(qn, kn, ek, src, dst)


def _sc_segsum_kernel(ex_hbm, src_hbm, out_hbm, srcv, exrows, padbuf, shared,
                      ss0, ss1, sx0, sx1):
    cid = lax.axis_index("c")
    sid = lax.axis_index("s")
    w = sid * NC + cid
    rows_per_sub = NPAD // NS
    ss = (ss0, ss1)
    sx = (sx0, sx1)

    def zrow(r, carry):
        for j in range(DIM // HD):
            padbuf[0, r, pl.ds(j * HD, HD)] = jnp.zeros((HD,), _f32)
            padbuf[1, r, pl.ds(j * HD, HD)] = jnp.zeros((HD,), _f32)
        return carry

    lax.fori_loop(0, CB3, zrow, 0)
    for t in range(rows_per_sub // CB3):
        pltpu.sync_copy(padbuf.at[0],
                        shared.at[pl.ds(sid * rows_per_sub + t * CB3, CB3)])
    plsc.subcore_barrier()

    def cbase(i):
        return w * EPW + jnp.minimum(i, CH3 - 1) * CB3

    def issue_io(i, b):
        base = cbase(i)
        pltpu.async_copy(src_hbm.at[pl.ds(base, CB3)], srcv.at[b], ss[b])
        pltpu.async_copy(ex_hbm.at[pl.ds(base, CB3)], exrows.at[b], sx[b])

    def wait_io(b):
        pltpu.make_async_copy(src_hbm.at[pl.ds(0, CB3)], srcv.at[b], ss[b]).wait()
        pltpu.make_async_copy(ex_hbm.at[pl.ds(0, CB3)], exrows.at[b], sx[b]).wait()

    issue_io(0, 0)
    issue_io(1, 1)

    def body2(i2, carry):
        for b in (0, 1):
            i = i2 * 2 + b
            wait_io(b)

            def fill(e, carry2):
                padbuf[b, e, pl.ds(0, HD)] = exrows[b, e]
                return carry2

            lax.fori_loop(0, CB3, fill, 0)
            pltpu.sync_copy(padbuf.at[b], shared.at[srcv.at[b]], add=True)
            issue_io(i + 2, b)
        return carry

    lax.fori_loop(0, CH3 // 2, body2, 0)
    wait_io(0)
    wait_io(1)
    plsc.subcore_barrier()
    for t in range(rows_per_sub // CB3):
        off = sid * rows_per_sub + t * CB3
        pltpu.sync_copy(shared.at[pl.ds(off, CB3)], out_hbm.at[cid].at[pl.ds(off, CB3)])


def _sc_segsum(ex16, src):
    kern = pl.kernel(
        _sc_segsum_kernel,
        out_type=jax.ShapeDtypeStruct((NC, NPAD, DIM), _f32),
        mesh=_mesh,
        compiler_params=pltpu.CompilerParams(needs_layout_passes=False),
        scratch_types=[
            pltpu.VMEM((2, CB3), jnp.int32),
            pltpu.VMEM((2, CB3, HD), _f32),
            pltpu.VMEM((2, CB3, DIM), _f32),
            pltpu.VMEM_SHARED((NPAD, DIM), _f32),
        ] + [pltpu.SemaphoreType.DMA] * 4,
    )
    return kern(ex16, src)


def _combine_body(p_ref, o_ref):
    o_ref[...] = p_ref[0] + p_ref[1]


def _tc_combine(parts):
    bn = 2048
    grid = (NPAD // bn,)
    return pl.pallas_call(
        _combine_body,
        grid=grid,
        in_specs=[pl.BlockSpec((2, bn, DIM), lambda i: (0, i, 0))],
        out_specs=pl.BlockSpec((bn, DIM), lambda i: (i, 0)),
        out_shape=jax.ShapeDtypeStruct((NPAD, DIM), _f32),
    )(parts)


def _sc_aggregate_kernel(vn_hbm, ev_hbm, ex_hbm, ssum_hbm, src_hbm, dst_hbm,
                         outp_hbm,
                         srcv, dstv, vrows, evrows, exrows, shared,
                         sv0, sv1, se0, se1, sx0, sx1, ss0, ss1, sd0, sd1):
    cid = lax.axis_index("c")
    sid = lax.axis_index("s")
    w = sid * NC + cid
    rows_per_sub = NPAD // NS
    sv = (sv0, sv1)
    se = (se0, se1)
    sx = (sx0, sx1)
    ss = (ss0, ss1)
    sd = (sd0, sd1)

    def zrow(r, carry):
        for j in range(DIM // HD):
            vrows[0, r, pl.ds(j * HD, HD)] = jnp.zeros((HD,), _f32)
        return carry

    lax.fori_loop(0, CB3, zrow, 0)
    for t in range(rows_per_sub // CB3):
        pltpu.sync_copy(vrows.at[0], shared.at[pl.ds(sid * rows_per_sub + t * CB3, CB3)])
    plsc.subcore_barrier()

    def cbase(i):
        return w * EPW + jnp.minimum(i, CH3 - 1) * CB3

    def issue_idx(i, b):
        base = cbase(i)
        pltpu.async_copy(src_hbm.at[pl.ds(base, CB3)], srcv.at[b], ss[b])
        pltpu.async_copy(dst_hbm.at[pl.ds(base, CB3)], dstv.at[b], sd[b])

    def wait_idx(b):
        pltpu.make_async_copy(src_hbm.at[pl.ds(0, CB3)], srcv.at[b], ss[b]).wait()
        pltpu.make_async_copy(dst_hbm.at[pl.ds(0, CB3)], dstv.at[b], sd[b]).wait()

    def issue_gathers(i, b):
        base = cbase(i)
        pltpu.async_copy(vn_hbm.at[dstv.at[b]], vrows.at[b], sv[b])
        pltpu.async_copy(ev_hbm.at[pl.ds(base, CB3)], evrows.at[b], se[b])
        pltpu.async_copy(ex_hbm.at[pl.ds(base, CB3)], exrows.at[b], sx[b])

    def wait_gathers(b):
        pltpu.make_async_copy(vn_hbm.at[pl.ds(0, CB3)], vrows.at[b], sv[b]).wait()
        pltpu.make_async_copy(ev_hbm.at[pl.ds(0, CB3)], evrows.at[b], se[b]).wait()
        pltpu.make_async_copy(ex_hbm.at[pl.ds(0, CB3)], exrows.at[b], sx[b]).wait()

    def compute(i, b):
        vr = vrows.at[b]
        er = evrows.at[b]
        xr = exrows.at[b]

        def edge_body(t, carry2):
            for u in range(2):
                e = t * 2 + u
                exv = xr[e]
                for h in range(HEADS):
                    a = exv[h]
                    sl = pl.ds(h * HD, HD)
                    vr[e, sl] = (vr[e, sl] + er[e, sl]) * a
            return carry2

        lax.fori_loop(0, CB3 // 2, edge_body, 0)
        pltpu.sync_copy(vr, shared.at[srcv.at[b]], add=True)

    issue_idx(0, 0)
    issue_idx(1, 1)
    wait_idx(0)
    issue_gathers(0, 0)

    def body2(i2, carry):
        for b in (0, 1):
            i = i2 * 2 + b
            wait_idx(1 - b)
            issue_gathers(i + 1, 1 - b)
            wait_gathers(b)
            compute(i, b)
            issue_idx(i + 2, b)
        return carry

    lax.fori_loop(0, CH3 // 2, body2, 0)
    wait_gathers(0)
    wait_idx(1)
    plsc.subcore_barrier()
    for t in range(rows_per_sub // CB3):
        off = sid * rows_per_sub + t * CB3
        pltpu.sync_copy(shared.at[pl.ds(off, CB3)], vrows.at[0])
        pltpu.sync_copy(ssum_hbm.at[pl.ds(off, CB3)], evrows.at[0])

        def nrow(r, carry):
            rec = 1.0 / (evrows[0, r, pl.ds(0, HD)] + 1e-16)
            for h in range(HEADS):
                sl = pl.ds(h * HD, HD)
                vrows[0, r, sl] = vrows[0, r, sl] * rec[h]
            return carry

        lax.fori_loop(0, CB3, nrow, 0)
        pltpu.sync_copy(vrows.at[0], outp_hbm.at[cid].at[pl.ds(off, CB3)])


def _sc_aggregate(vn, ev, ex16, ssum, src, dst):
    kern = pl.kernel(
        _sc_aggregate_kernel,
        out_type=jax.ShapeDtypeStruct((NC, NPAD, DIM), _f32),
        mesh=_mesh,
        compiler_params=pltpu.CompilerParams(needs_layout_passes=False),
        scratch_types=[
            pltpu.VMEM((2, CB3), jnp.int32),
            pltpu.VMEM((2, CB3), jnp.int32),
            pltpu.VMEM((2, CB3, DIM), _f32),
            pltpu.VMEM((2, CB3, DIM), _f32),
            pltpu.VMEM((2, CB3, HD), _f32),
            pltpu.VMEM_SHARED((NPAD, DIM), _f32),
        ] + [pltpu.SemaphoreType.DMA] * 10,
    )
    return kern(vn, ev, ex16, ssum, src, dst)


CBA = 128
NCHA = E // CBA
CHA = 80


def _sc_attn_kernel(ex_hbm, ssum_hbm, src_hbm, attn_hbm,
                    srcv, srows, exrows, attn_st,
                    ss0, ss1, sr0, sr1, sx0, sx1, so0, so1):
    w = lax.axis_index("s") * NC + lax.axis_index("c")
    ss = (ss0, ss1)
    sr = (sr0, sr1)
    sx = (sx0, sx1)
    so = (so0, so1)

    def cbase(i):
        return lax.rem(w + i * NW, NCHA) * CBA

    def issue_idx(i, b):
        pltpu.async_copy(src_hbm.at[pl.ds(cbase(i), CBA)], srcv.at[b], ss[b])

    def wait_idx(b):
        pltpu.make_async_copy(src_hbm.at[pl.ds(0, CBA)], srcv.at[b], ss[b]).wait()

    def issue_gathers(i, b):
        pltpu.async_copy(ssum_hbm.at[srcv.at[b]], srows.at[b], sr[b])
        pltpu.async_copy(ex_hbm.at[pl.ds(cbase(i), CBA)], exrows.at[b], sx[b])

    def wait_gathers(b):
        pltpu.make_async_copy(ssum_hbm.at[pl.ds(0, CBA)], srows.at[b], sr[b]).wait()
        pltpu.make_async_copy(ex_hbm.at[pl.ds(0, CBA)], exrows.at[b], sx[b]).wait()

    def wait_out(b):
        pltpu.make_async_copy(attn_st.at[b], attn_hbm.at[pl.ds(0, CBA)], so[b]).wait()

    def compute(i, b):
        sb = srows.at[b]
        xb = exrows.at[b]
        ab = attn_st.at[b]

        def edge_body(e, carry2):
            ab[e] = xb[e] / (sb[e, pl.ds(0, HD)] + 1e-16)
            return carry2

        lax.fori_loop(0, CBA, edge_body, 0)
        pltpu.async_copy(ab, attn_hbm.at[pl.ds(cbase(i), CBA)], so[b])

    issue_idx(0, 0)
    issue_idx(1, 1)
    wait_idx(0)
    issue_gathers(0, 0)

    def body2(i2, carry):
        for b in (0, 1):
            i = i2 * 2 + b
            wait_gathers(b)
            issue_idx(i + 2, b)
            wait_idx(1 - b)
            issue_gathers(i + 1, 1 - b)

            @pl.when(i >= 2)
            def _():
                wait_out(b)

            compute(i, b)
        return carry

    lax.fori_loop(0, CHA // 2, body2, 0)
    wait_gathers(0)
    wait_idx(1)
    wait_out(0)
    wait_out(1)


def _sc_attn(ex16, ssum, src):
    kern = pl.kernel(
        _sc_attn_kernel,
        out_type=jax.ShapeDtypeStruct((E, HD), _f32),
        mesh=_mesh,
        compiler_params=pltpu.CompilerParams(needs_layout_passes=False),
        scratch_types=[
            pltpu.VMEM((2, CBA), jnp.int32),
            pltpu.VMEM((2, CBA, DIM), _f32),
            pltpu.VMEM((2, CBA, HD), _f32),
            pltpu.VMEM((2, CBA, HD), _f32),
        ] + [pltpu.SemaphoreType.DMA] * 8,
    )
    return kern(ex16, ssum, src)


def kernel(x, edges, edge_index, Wq, Wk, Wv, Wek, Wev, Wexp, Wsq, Wout, bout):
    src = edge_index[0]
    dst = edge_index[1]

    wq_s = Wq * SCALE
    pexp = jnp.zeros((HD, HD), _f32).at[:HEADS, :HEADS].set(Wexp.T.astype(_f32))
    psq = jnp.zeros((HD, HD), _f32).at[:HEADS, :HEADS].set(Wsq.T.astype(_f32))
    eye8 = jnp.eye(HEADS, dtype=_f32)
    bexp = jnp.kron(eye8, pexp)
    bsq = jnp.kron(eye8, psq)

    qn, kn, vn = _tc_node_proj(x, wq_s, Wk, Wv)
    ek, ev = _tc_edge_proj(edges, Wek, Wev)

    logits = _sc_logits(qn, kn, ek, src, dst)
    g_packed = logits.reshape(E // 8, DIM)
    ex_packed = _tc_head_mlp(g_packed, bexp, bsq)
    ex16 = ex_packed.reshape(E, HD)

    ssum_p = _sc_segsum(ex16, src)
    ssum = _tc_combine(ssum_p)
    attn16 = _sc_attn(ex16, ssum, src)
    out_p = _sc_aggregate(vn, ev, ex16, ssum, src, dst)

    out = _tc_out_proj(out_p, Wout, bout.reshape(1, DIM))
    attn_he = attn16[:, :HEADS].T
    return out, attn_he


# speedup vs baseline: 1.0526x; 1.0019x over previous
import functools

import jax
import jax.numpy as jnp
from jax import lax
from jax.experimental import pallas as pl
from jax.experimental.pallas import tpu as pltpu
from jax.experimental.pallas import tpu_sc as plsc

N = 10000
NPAD = 10240
E = 320000
DIM = 128
HEADS = 8
HD = 16
SCALE = HD ** -0.5

NC = 2
NS = 16
NW = NC * NS
CB = 128
NCH = E // CB
CH_PER_W = -(-NCH // NW)
CB3 = 40
EPW = E // NW
CH3 = EPW // CB3

_f32 = jnp.float32
_mesh = plsc.VectorSubcoreMesh(core_axis_name="c", subcore_axis_name="s")


def _matmul_t(a, w):
    return lax.dot_general(a, w, (((1,), (1,)), ((), ())),
                           preferred_element_type=_f32)


def _node_proj_body(x_ref, wq_ref, wk_ref, wv_ref, q_ref, k_ref, v_ref):
    xb = x_ref[...]
    q_ref[...] = _matmul_t(xb, wq_ref[...])
    k_ref[...] = _matmul_t(xb, wk_ref[...])
    v_ref[...] = _matmul_t(xb, wv_ref[...])


def _tc_node_proj(x, wq_s, wk, wv):
    bn = 2000
    grid = (N // bn,)
    bspec = pl.BlockSpec((bn, DIM), lambda i: (i, 0))
    wspec = pl.BlockSpec((DIM, DIM), lambda i: (0, 0))
    out = jax.ShapeDtypeStruct((N, DIM), _f32)
    return pl.pallas_call(
        _node_proj_body,
        grid=grid,
        in_specs=[bspec, wspec, wspec, wspec],
        out_specs=[bspec, bspec, bspec],
        out_shape=[out, out, out],
    )(x, wq_s, wk, wv)


def _edge_proj_body(e_ref, wek_ref, wev_ref, ek_ref, ev_ref):
    eb = e_ref[...]
    ek_ref[...] = _matmul_t(eb, wek_ref[...])
    ev_ref[...] = _matmul_t(eb, wev_ref[...])


def _tc_edge_proj(edges, wek, wev):
    be = 10000
    grid = (E // be,)
    bspec = pl.BlockSpec((be, DIM), lambda i: (i, 0))
    wspec = pl.BlockSpec((DIM, DIM), lambda i: (0, 0))
    out = jax.ShapeDtypeStruct((E, DIM), _f32)
    return pl.pallas_call(
        _edge_proj_body,
        grid=grid,
        in_specs=[bspec, wspec, wspec],
        out_specs=[bspec, bspec],
        out_shape=[out, out],
    )(edges, wek, wev)


def _head_mlp_body(g_ref, bexp_ref, bsq_ref, ex_ref):
    g = g_ref[...]
    g = jnp.dot(g, bexp_ref[...], preferred_element_type=_f32)
    g = 0.5 * g * (1.0 + lax.erf(g * (2.0 ** -0.5)))
    g = jnp.dot(g, bsq_ref[...], preferred_element_type=_f32)
    ex_ref[...] = jnp.exp(g)


def _tc_head_mlp(g_packed, bexp, bsq):
    rows = E // 8
    br = 2000
    grid = (rows // br,)
    bspec = pl.BlockSpec((br, DIM), lambda i: (i, 0))
    wspec = pl.BlockSpec((DIM, DIM), lambda i: (0, 0))
    return pl.pallas_call(
        _head_mlp_body,
        grid=grid,
        in_specs=[bspec, wspec, wspec],
        out_specs=bspec,
        out_shape=jax.ShapeDtypeStruct((rows, DIM), _f32),
    )(g_packed, bexp, bsq)


def _out_proj_body(p_ref, w_ref, b_ref, o_ref):
    acc = p_ref[0] + p_ref[1]
    o_ref[...] = _matmul_t(acc, w_ref[...]) + b_ref[...]


def _tc_out_proj(parts, wout, bout2d):
    bn = 2000
    grid = (N // bn,)
    return pl.pallas_call(
        _out_proj_body,
        grid=grid,
        in_specs=[
            pl.BlockSpec((2, bn, DIM), lambda i: (0, i, 0)),
            pl.BlockSpec((DIM, DIM), lambda i: (0, 0)),
            pl.BlockSpec((1, DIM), lambda i: (0, 0)),
        ],
        out_specs=pl.BlockSpec((bn, DIM), lambda i: (i, 0)),
        out_shape=jax.ShapeDtypeStruct((N, DIM), _f32),
    )(parts, wout, bout2d)


CB1 = 80
NCH1 = E // CB1
CH1 = 126


def _sc_logits_kernel(qn_hbm, kn_hbm, ek_hbm, src_hbm, dst_hbm, out_hbm,
                      srcv, dstv, qrows, krows, ekrows, lg, fbuf,
                      sq0, sq1, sk0, sk1, se0, se1,
                      ss0, ss1, sd0, sd1, so0, so1):
    w = lax.axis_index("s") * NC + lax.axis_index("c")
    l16 = lax.iota(jnp.int32, HD)
    i0 = ((l16 >> 3) * 16) + (l16 & 7)
    i1 = 128 + ((l16 >> 2) * 8) + (l16 & 3)
    i2 = 192 + ((l16 >> 1) * 4) + (l16 & 1)
    i3 = 224 + ((l16 & 7) * 2)
    sq = (sq0, sq1)
    sk = (sk0, sk1)
    se = (se0, se1)
    ss = (ss0, ss1)
    sd = (sd0, sd1)
    so = (so0, so1)

    def cbase(i):
        return lax.rem(w + i * NW, NCH1) * CB1

    def issue_idx(i, b):
        base = cbase(i)
        pltpu.async_copy(src_hbm.at[pl.ds(base, CB1)], srcv.at[b], ss[b])
        pltpu.async_copy(dst_hbm.at[pl.ds(base, CB1)], dstv.at[b], sd[b])

    def wait_idx(b):
        pltpu.make_async_copy(src_hbm.at[pl.ds(0, CB1)], srcv.at[b], ss[b]).wait()
        pltpu.make_async_copy(dst_hbm.at[pl.ds(0, CB1)], dstv.at[b], sd[b]).wait()

    def issue_gathers(i, b):
        base = cbase(i)
        pltpu.async_copy(qn_hbm.at[srcv.at[b]], qrows.at[b], sq[b])
        pltpu.async_copy(kn_hbm.at[dstv.at[b]], krows.at[b], sk[b])
        pltpu.async_copy(ek_hbm.at[pl.ds(base, CB1)], ekrows.at[b], se[b])

    def wait_gathers(b):
        pltpu.make_async_copy(qn_hbm.at[pl.ds(0, CB1)], qrows.at[b], sq[b]).wait()
        pltpu.make_async_copy(kn_hbm.at[pl.ds(0, CB1)], krows.at[b], sk[b]).wait()
        pltpu.make_async_copy(ek_hbm.at[pl.ds(0, CB1)], ekrows.at[b], se[b]).wait()

    def wait_out(b):
        pltpu.make_async_copy(lg.at[b], out_hbm.at[pl.ds(0, CB1)], so[b]).wait()

    def compute(i, b):
        qr = qrows.at[b]
        kr = krows.at[b]
        er = ekrows.at[b]
        lgb = lg.at[b]

        def edge_body(t, carry2):
            for u in range(4):
                e = t * 4 + u
                fb = u * 256
                for h in range(HEADS):
                    sl = pl.ds(h * HD, HD)
                    fbuf[pl.ds(fb + h * HD, HD)] = (
                        qr[e, sl] * (kr[e, sl] + er[e, sl]))
            for u in range(4):
                fb = u * 256
                for g in range(4):
                    v = (plsc.load_gather(fbuf, [fb + 32 * g + i0])
                         + plsc.load_gather(fbuf, [fb + 32 * g + i0 + 8]))
                    fbuf[pl.ds(fb + 128 + 16 * g, HD)] = v
                for m in range(2):
                    v = (plsc.load_gather(fbuf, [fb + 32 * m + i1])
                         + plsc.load_gather(fbuf, [fb + 32 * m + i1 + 4]))
                    fbuf[pl.ds(fb + 192 + 16 * m, HD)] = v
                v = (plsc.load_gather(fbuf, [fb + i2])
                     + plsc.load_gather(fbuf, [fb + i2 + 2]))
                fbuf[pl.ds(fb + 224, HD)] = v
                lgb[t * 4 + u] = (plsc.load_gather(fbuf, [fb + i3])
                                  + plsc.load_gather(fbuf, [fb + i3 + 1]))
            return carry2

        lax.fori_loop(0, CB1 // 4, edge_body, 0)
        pltpu.async_copy(lgb, out_hbm.at[pl.ds(cbase(i), CB1)], so[b])

    issue_idx(0, 0)
    issue_idx(1, 1)
    wait_idx(0)
    issue_gathers(0, 0)

    def body2(i2, carry):
        for b in (0, 1):
            i = i2 * 2 + b
            wait_gathers(b)
            issue_idx(i + 2, b)
            wait_idx(1 - b)
            issue_gathers(i + 1, 1 - b)

            @pl.when(i >= 2)
            def _():
                wait_out(b)

            compute(i, b)
        return carry

    lax.fori_loop(0, CH1 // 2, body2, 0)
    wait_gathers(0)
    wait_idx(1)
    wait_out(0)
    wait_out(1)


def _sc_logits(qn, kn, ek, src, dst):
    kern = pl.kernel(
        _sc_logits_kernel,
        out_type=jax.ShapeDtypeStruct((E, HD), _f32),
        mesh=_mesh,
        compiler_params=pltpu.CompilerParams(needs_layout_passes=False),
        scratch_types=[
            pltpu.VMEM((2, CB1), jnp.int32),
            pltpu.VMEM((2, CB1), jnp.int32),
            pltpu.VMEM((2, CB1, DIM), _f32),
            pltpu.VMEM((2, CB1, DIM), _f32),
            pltpu.VMEM((2, CB1, DIM), _f32),
            pltpu.VMEM((2, CB1, HD), _f32),
            pltpu.VMEM((1024,), _f32),
        ] + [pltpu.SemaphoreType.DMA] * 12,
    )
    return kern(qn, kn, ek, src, dst)


def _sc_segsum_kernel(ex_hbm, src_hbm, out_hbm, srcv, exrows, padbuf, shared,
                      ss0, ss1, sx0, sx1):
    cid = lax.axis_index("c")
    sid = lax.axis_index("s")
    w = sid * NC + cid
    rows_per_sub = NPAD // NS
    ss = (ss0, ss1)
    sx = (sx0, sx1)

    def zrow(r, carry):
        for j in range(DIM // HD):
            padbuf[0, r, pl.ds(j * HD, HD)] = jnp.zeros((HD,), _f32)
            padbuf[1, r, pl.ds(j * HD, HD)] = jnp.zeros((HD,), _f32)
        return carry

    lax.fori_loop(0, CB3, zrow, 0)
    for t in range(rows_per_sub // CB3):
        pltpu.sync_copy(padbuf.at[0],
                        shared.at[pl.ds(sid * rows_per_sub + t * CB3, CB3)])
    plsc.subcore_barrier()

    def cbase(i):
        return w * EPW + jnp.minimum(i, CH3 - 1) * CB3

    def issue_io(i, b):
        base = cbase(i)
        pltpu.async_copy(src_hbm.at[pl.ds(base, CB3)], srcv.at[b], ss[b])
        pltpu.async_copy(ex_hbm.at[pl.ds(base, CB3)], exrows.at[b], sx[b])

    def wait_io(b):
        pltpu.make_async_copy(src_hbm.at[pl.ds(0, CB3)], srcv.at[b], ss[b]).wait()
        pltpu.make_async_copy(ex_hbm.at[pl.ds(0, CB3)], exrows.at[b], sx[b]).wait()

    issue_io(0, 0)
    issue_io(1, 1)

    def body2(i2, carry):
        for b in (0, 1):
            i = i2 * 2 + b
            wait_io(b)

            def fill(e, carry2):
                padbuf[b, e, pl.ds(0, HD)] = exrows[b, e]
                return carry2

            lax.fori_loop(0, CB3, fill, 0)
            pltpu.sync_copy(padbuf.at[b], shared.at[srcv.at[b]], add=True)
            issue_io(i + 2, b)
        return carry

    lax.fori_loop(0, CH3 // 2, body2, 0)
    wait_io(0)
    wait_io(1)
    plsc.subcore_barrier()
    for t in range(rows_per_sub // CB3):
        off = sid * rows_per_sub + t * CB3
        pltpu.sync_copy(shared.at[pl.ds(off, CB3)], out_hbm.at[cid].at[pl.ds(off, CB3)])


def _sc_segsum(ex16, src):
    kern = pl.kernel(
        _sc_segsum_kernel,
        out_type=jax.ShapeDtypeStruct((NC, NPAD, DIM), _f32),
        mesh=_mesh,
        compiler_params=pltpu.CompilerParams(needs_layout_passes=False),
        scratch_types=[
            pltpu.VMEM((2, CB3), jnp.int32),
            pltpu.VMEM((2, CB3, HD), _f32),
            pltpu.VMEM((2, CB3, DIM), _f32),
            pltpu.VMEM_SHARED((NPAD, DIM), _f32),
        ] + [pltpu.SemaphoreType.DMA] * 4,
    )
    return kern(ex16, src)


def _combine_body(p_ref, o_ref):
    o_ref[...] = p_ref[0] + p_ref[1]


def _tc_combine(parts):
    bn = 2048
    grid = (NPAD // bn,)
    return pl.pallas_call(
        _combine_body,
        grid=grid,
        in_specs=[pl.BlockSpec((2, bn, DIM), lambda i: (0, i, 0))],
        out_specs=pl.BlockSpec((bn, DIM), lambda i: (i, 0)),
        out_shape=jax.ShapeDtypeStruct((NPAD, DIM), _f32),
    )(parts)


def _sc_aggregate_kernel(vn_hbm, ev_hbm, ex_hbm, ssum_hbm, src_hbm, dst_hbm,
                         outp_hbm,
                         srcv, dstv, vrows, evrows, exrows, shared,
                         sv0, sv1, se0, se1, sx0, sx1, ss0, ss1, sd0, sd1):
    cid = lax.axis_index("c")
    sid = lax.axis_index("s")
    w = sid * NC + cid
    rows_per_sub = NPAD // NS
    sv = (sv0, sv1)
    se = (se0, se1)
    sx = (sx0, sx1)
    ss = (ss0, ss1)
    sd = (sd0, sd1)

    def zrow(r, carry):
        for j in range(DIM // HD):
            vrows[0, r, pl.ds(j * HD, HD)] = jnp.zeros((HD,), _f32)
        return carry

    lax.fori_loop(0, CB3, zrow, 0)
    for t in range(rows_per_sub // CB3):
        pltpu.sync_copy(vrows.at[0], shared.at[pl.ds(sid * rows_per_sub + t * CB3, CB3)])
    plsc.subcore_barrier()

    def cbase(i):
        return w * EPW + jnp.minimum(i, CH3 - 1) * CB3

    def issue_idx(i, b):
        base = cbase(i)
        pltpu.async_copy(src_hbm.at[pl.ds(base, CB3)], srcv.at[b], ss[b])
        pltpu.async_copy(dst_hbm.at[pl.ds(base, CB3)], dstv.at[b], sd[b])

    def wait_idx(b):
        pltpu.make_async_copy(src_hbm.at[pl.ds(0, CB3)], srcv.at[b], ss[b]).wait()
        pltpu.make_async_copy(dst_hbm.at[pl.ds(0, CB3)], dstv.at[b], sd[b]).wait()

    def issue_gathers(i, b):
        base = cbase(i)
        pltpu.async_copy(vn_hbm.at[dstv.at[b]], vrows.at[b], sv[b])
        pltpu.async_copy(ev_hbm.at[pl.ds(base, CB3)], evrows.at[b], se[b])
        pltpu.async_copy(ex_hbm.at[pl.ds(base, CB3)], exrows.at[b], sx[b])

    def wait_gathers(b):
        pltpu.make_async_copy(vn_hbm.at[pl.ds(0, CB3)], vrows.at[b], sv[b]).wait()
        pltpu.make_async_copy(ev_hbm.at[pl.ds(0, CB3)], evrows.at[b], se[b]).wait()
        pltpu.make_async_copy(ex_hbm.at[pl.ds(0, CB3)], exrows.at[b], sx[b]).wait()

    def compute(i, b):
        vr = vrows.at[b]
        er = evrows.at[b]
        xr = exrows.at[b]

        def edge_body(t, carry2):
            for u in range(2):
                e = t * 2 + u
                exv = xr[e]
                for h in range(HEADS):
                    a = exv[h]
                    sl = pl.ds(h * HD, HD)
                    vr[e, sl] = (vr[e, sl] + er[e, sl]) * a
            return carry2

        lax.fori_loop(0, CB3 // 2, edge_body, 0)
        pltpu.sync_copy(vr, shared.at[srcv.at[b]], add=True)

    issue_idx(0, 0)
    issue_idx(1, 1)
    wait_idx(0)
    issue_gathers(0, 0)

    def body2(i2, carry):
        for b in (0, 1):
            i = i2 * 2 + b
            wait_idx(1 - b)
            issue_gathers(i + 1, 1 - b)
            wait_gathers(b)
            compute(i, b)
            issue_idx(i + 2, b)
        return carry

    lax.fori_loop(0, CH3 // 2, body2, 0)
    wait_gathers(0)
    wait_idx(1)
    plsc.subcore_barrier()
    for t in range(rows_per_sub // CB3):
        off = sid * rows_per_sub + t * CB3
        pltpu.sync_copy(shared.at[pl.ds(off, CB3)], vrows.at[0])
        pltpu.sync_copy(ssum_hbm.at[pl.ds(off, CB3)], evrows.at[0])

        def nrow(r, carry):
            rec = 1.0 / (evrows[0, r, pl.ds(0, HD)] + 1e-16)
            for h in range(HEADS):
                sl = pl.ds(h * HD, HD)
                vrows[0, r, sl] = vrows[0, r, sl] * rec[h]
            return carry

        lax.fori_loop(0, CB3, nrow, 0)
        pltpu.sync_copy(vrows.at[0], outp_hbm.at[cid].at[pl.ds(off, CB3)])


def _sc_aggregate(vn, ev, ex16, ssum, src, dst):
    kern = pl.kernel(
        _sc_aggregate_kernel,
        out_type=jax.ShapeDtypeStruct((NC, NPAD, DIM), _f32),
        mesh=_mesh,
        compiler_params=pltpu.CompilerParams(needs_layout_passes=False),
        scratch_types=[
            pltpu.VMEM((2, CB3), jnp.int32),
            pltpu.VMEM((2, CB3), jnp.int32),
            pltpu.VMEM((2, CB3, DIM), _f32),
            pltpu.VMEM((2, CB3, DIM), _f32),
            pltpu.VMEM((2, CB3, HD), _f32),
            pltpu.VMEM_SHARED((NPAD, DIM), _f32),
        ] + [pltpu.SemaphoreType.DMA] * 10,
    )
    return kern(vn, ev, ex16, ssum, src, dst)


CBA = 128
NCHA = E // CBA
CHA = 80


def _sc_attn_kernel(ex_hbm, ssum_hbm, src_hbm, attn_hbm,
                    srcv, srows, exrows, attn_st,
                    ss0, ss1, sr0, sr1, sx0, sx1, so0, so1):
    w = lax.axis_index("s") * NC + lax.axis_index("c")
    ss = (ss0, ss1)
    sr = (sr0, sr1)
    sx = (sx0, sx1)
    so = (so0, so1)

    def cbase(i):
        return lax.rem(w + i * NW, NCHA) * CBA

    def issue_idx(i, b):
        pltpu.async_copy(src_hbm.at[pl.ds(cbase(i), CBA)], srcv.at[b], ss[b])

    def wait_idx(b):
        pltpu.make_async_copy(src_hbm.at[pl.ds(0, CBA)], srcv.at[b], ss[b]).wait()

    def issue_gathers(i, b):
        pltpu.async_copy(ssum_hbm.at[srcv.at[b]], srows.at[b], sr[b])
        pltpu.async_copy(ex_hbm.at[pl.ds(cbase(i), CBA)], exrows.at[b], sx[b])

    def wait_gathers(b):
        pltpu.make_async_copy(ssum_hbm.at[pl.ds(0, CBA)], srows.at[b], sr[b]).wait()
        pltpu.make_async_copy(ex_hbm.at[pl.ds(0, CBA)], exrows.at[b], sx[b]).wait()

    def wait_out(b):
        pltpu.make_async_copy(attn_st.at[b], attn_hbm.at[pl.ds(0, CBA)], so[b]).wait()

    def compute(i, b):
        sb = srows.at[b]
        xb = exrows.at[b]
        ab = attn_st.at[b]

        def edge_body(e, carry2):
            ab[e] = xb[e] / (sb[e, pl.ds(0, HD)] + 1e-16)
            return carry2

        lax.fori_loop(0, CBA, edge_body, 0)
        pltpu.async_copy(ab, attn_hbm.at[pl.ds(cbase(i), CBA)], so[b])

    issue_idx(0, 0)
    issue_idx(1, 1)
    wait_idx(0)
    issue_gathers(0, 0)

    def body2(i2, carry):
        for b in (0, 1):
            i = i2 * 2 + b
            wait_gathers(b)
            issue_idx(i + 2, b)
            wait_idx(1 - b)
            issue_gathers(i + 1, 1 - b)

            @pl.when(i >= 2)
            def _():
                wait_out(b)

            compute(i, b)
        return carry

    lax.fori_loop(0, CHA // 2, body2, 0)
    wait_gathers(0)
    wait_idx(1)
    wait_out(0)
    wait_out(1)


def _sc_attn(ex16, ssum, src):
    kern = pl.kernel(
        _sc_attn_kernel,
        out_type=jax.ShapeDtypeStruct((E, HD), _f32),
        mesh=_mesh,
        compiler_params=pltpu.CompilerParams(needs_layout_passes=False),
        scratch_types=[
            pltpu.VMEM((2, CBA), jnp.int32),
            pltpu.VMEM((2, CBA, DIM), _f32),
            pltpu.VMEM((2, CBA, HD), _f32),
            pltpu.VMEM((2, CBA, HD), _f32),
        ] + [pltpu.SemaphoreType.DMA] * 8,
    )
    return kern(ex16, ssum, src)


def kernel(x, edges, edge_index, Wq, Wk, Wv, Wek, Wev, Wexp, Wsq, Wout, bout):
    src = edge_index[0]
    dst = edge_index[1]

    wq_s = Wq * SCALE
    pexp = jnp.zeros((HD, HD), _f32).at[:HEADS, :HEADS].set(Wexp.T.astype(_f32))
    psq = jnp.zeros((HD, HD), _f32).at[:HEADS, :HEADS].set(Wsq.T.astype(_f32))
    eye8 = jnp.eye(HEADS, dtype=_f32)
    bexp = jnp.kron(eye8, pexp)
    bsq = jnp.kron(eye8, psq)

    qn, kn, vn = _tc_node_proj(x, wq_s, Wk, Wv)
    ek, ev = _tc_edge_proj(edges, Wek, Wev)

    logits = _sc_logits(qn, kn, ek, src, dst)
    g_packed = logits.reshape(E // 8, DIM)
    ex_packed = _tc_head_mlp(g_packed, bexp, bsq)
    ex16 = ex_packed.reshape(E, HD)

    ssum_p = _sc_segsum(ex16, src)
    ssum = _tc_combine(ssum_p)
    attn16 = _sc_attn(ex16, ssum, src)
    out_p = _sc_aggregate(vn, ev, ex16, ssum, src, dst)

    out = _tc_out_proj(out_p, Wout, bout.reshape(1, DIM))
    attn_he = attn16[:, :HEADS].T
    return out, attn_he
